# baseline (speedup 1.0000x reference)
import jax
import jax.numpy as jnp
from jax.experimental import pallas as pl

_B = 128
_L = 512
_NSEG = 50
_VOCAB = 3001


_SB = 10
_GRID = _NSEG // _SB


def _fill_kernel(out_ref, mask_ref, ib_ref):
    i = pl.program_id(0)
    out_ref[...] = jnp.zeros(out_ref.shape, jnp.float32)
    out_ref[:, pl.ds(0, 1), :] = jnp.ones((_SB, 1, _B), jnp.float32)

    @pl.when(i == 0)
    def _():
        out_ref[0, pl.ds(0, 2), pl.ds(0, 1)] = jax.lax.broadcasted_iota(
            jnp.int32, (2, 1), 0).astype(jnp.float32)
        mask_ref[...] = jnp.ones(mask_ref.shape, jnp.float32)
        ib_ref[...] = jnp.ones(ib_ref.shape, jnp.float32)


def kernel(x):
    del x
    out_t, mask_t, ib_t = pl.pallas_call(
        _fill_kernel,
        grid=(_GRID,),
        out_specs=[
            pl.BlockSpec((_SB, _VOCAB, _B), lambda i: (i, 0, 0)),
            pl.BlockSpec((_NSEG, _B), lambda i: (0, 0)),
            pl.BlockSpec((_L + 1, _B), lambda i: (0, 0)),
        ],
        out_shape=[
            jax.ShapeDtypeStruct((_NSEG, _VOCAB, _B), jnp.float32),
            jax.ShapeDtypeStruct((_NSEG, _B), jnp.float32),
            jax.ShapeDtypeStruct((_L + 1, _B), jnp.float32),
        ],
    )()
    out = jnp.transpose(out_t, (2, 0, 1))
    mask = jnp.transpose(mask_t, (1, 0))
    in_boundary = jnp.transpose(ib_t, (1, 0))
    return (out, mask, in_boundary)
